# baseline (speedup 1.0000x reference)
import functools

import jax
import jax.numpy as jnp
from jax import lax
from jax.experimental import pallas as pl
from jax.experimental.pallas import tpu as pltpu
from jax.experimental.pallas import tpu_sc as plsc

D = 64
DP = 128
V = 1000000
BATCH = 16384
HIST = 50
B = BATCH * HIST
NC, NS = 2, 16
NW = NC * NS
B_PER_W = B // NW
E_PER_W = B_PER_W // HIST

E_PER_CHUNK = 4
CHUNK = E_PER_CHUNK * HIST
N_CHUNKS = B_PER_W // CHUNK
NB = 4
FL = NB // 2
G_OUTER = N_CHUNKS // NB

PAD_BLK = 2048

_mesh = plsc.VectorSubcoreMesh(core_axis_name="c", subcore_axis_name="s")
_params = pltpu.CompilerParams(use_tc_tiling_on_sc=True)


def _pad_body(tab_t_ref, out_ref):
    eye = (lax.broadcasted_iota(jnp.int32, (D, D), 0)
           == lax.broadcasted_iota(jnp.int32, (D, D), 1)).astype(jnp.float32)
    out_ref[:, :D] = lax.dot_general(
        tab_t_ref[...], eye,
        dimension_numbers=(((0,), (0,)), ((), ())),
        preferred_element_type=jnp.float32)


_pad_table_t = pl.pallas_call(
    _pad_body,
    grid=(pl.cdiv(V, PAD_BLK),),
    in_specs=[pl.BlockSpec((D, PAD_BLK), lambda i: (0, i))],
    out_specs=pl.BlockSpec((PAD_BLK, DP), lambda i: (i, 0)),
    out_shape=jax.ShapeDtypeStruct((V, DP), jnp.float32),
)


@functools.partial(
    pl.kernel,
    mesh=_mesh,
    compiler_params=_params,
    out_type=jax.ShapeDtypeStruct((BATCH, HIST, DP), jnp.float32),
    scratch_types=[
        pltpu.VMEM((B_PER_W,), jnp.int32),
        pltpu.VMEM((NB, CHUNK, DP), jnp.float32),
        pltpu.SemaphoreType.DMA,
        pltpu.SemaphoreType.DMA,
        pltpu.SemaphoreType.DMA,
        pltpu.SemaphoreType.DMA,
        pltpu.SemaphoreType.DMA,
        pltpu.SemaphoreType.DMA,
        pltpu.SemaphoreType.DMA,
        pltpu.SemaphoreType.DMA,
    ],
)
def _gather(table_hbm, idx_hbm, out_hbm, idx_v, rows_v,
            sg0, sg1, sg2, sg3, so0, so1, so2, so3):
    sem_g = [sg0, sg1, sg2, sg3]
    sem_o = [so0, so1, so2, so3]
    wid = lax.axis_index("s") * NC + lax.axis_index("c")
    base = wid * B_PER_W
    ebase = wid * E_PER_W

    pltpu.sync_copy(idx_hbm.at[pl.ds(base, B_PER_W)], idx_v)

    def g_start(i, b):
        idx_slice = idx_v.at[pl.ds(i * CHUNK, CHUNK)]
        pltpu.async_copy(table_hbm.at[idx_slice], rows_v.at[b], sem_g[b])

    def g_wait(i, b):
        idx_slice = idx_v.at[pl.ds(i * CHUNK, CHUNK)]
        pltpu.make_async_copy(table_hbm.at[idx_slice], rows_v.at[b], sem_g[b]).wait()

    def o_start(i, b):
        e0 = ebase + i * E_PER_CHUNK
        for e in range(E_PER_CHUNK):
            pltpu.async_copy(rows_v.at[b].at[pl.ds(e * HIST, HIST)],
                             out_hbm.at[e0 + e], sem_o[b])

    def o_wait(i, b):
        e0 = ebase + i * E_PER_CHUNK
        for e in range(E_PER_CHUNK):
            pltpu.make_async_copy(rows_v.at[b].at[pl.ds(e * HIST, HIST)],
                                  out_hbm.at[e0 + e], sem_o[b]).wait()

    for b in range(FL):
        g_start(b, b)
    for b in range(NB):
        g_wait(b, b)
        o_start(b, b)
        j = b + FL
        bj = j % NB
        if j >= NB:
            o_wait(j - NB, bj)
        g_start(j, bj)

    def body(g, carry):
        i0 = g * NB
        for b in range(NB):
            i = i0 + b
            g_wait(i, b)
            o_start(i, b)
            j = i + FL
            bj = (b + FL) % NB
            o_wait(j - NB, bj)
            g_start(j, bj)
        return carry

    lax.fori_loop(1, G_OUTER - 1, body, 0)

    i0 = (G_OUTER - 1) * NB
    for b in range(NB):
        i = i0 + b
        g_wait(i, b)
        o_start(i, b)
        j = i + FL
        if j < N_CHUNKS:
            bj = (b + FL) % NB
            o_wait(j - NB, bj)
            g_start(j, bj)
    for b in range(NB):
        o_wait(N_CHUNKS - NB + b, b)


def kernel(idx, table):
    tab_padded = _pad_table_t(table.T)
    out = _gather(tab_padded, idx.reshape(-1))
    return out[:, :, :D]


# speedup vs baseline: 1.2834x; 1.2834x over previous
import functools

import jax
import jax.numpy as jnp
from jax import lax
from jax.experimental import pallas as pl
from jax.experimental.pallas import tpu as pltpu
from jax.experimental.pallas import tpu_sc as plsc

D = 64
DP = 128
V = 1000000
BATCH = 16384
HIST = 50
B = BATCH * HIST
NC, NS = 2, 16
NW = NC * NS
B_PER_W = B // NW
E_PER_W = B_PER_W // HIST

E_PER_CHUNK = 4
CHUNK = E_PER_CHUNK * HIST
N_CHUNKS = B_PER_W // CHUNK
NB = 4
FL = NB // 2
G_OUTER = N_CHUNKS // NB

PAD_BLK = 8192

_mesh = plsc.VectorSubcoreMesh(core_axis_name="c", subcore_axis_name="s")
_params = pltpu.CompilerParams(use_tc_tiling_on_sc=True)


def _pad_body(tab_t_ref, out_ref):
    out_ref[:, :D] = tab_t_ref[...].T


_pad_table_t = pl.pallas_call(
    _pad_body,
    grid=(pl.cdiv(V, PAD_BLK),),
    in_specs=[pl.BlockSpec((D, PAD_BLK), lambda i: (0, i))],
    out_specs=pl.BlockSpec((PAD_BLK, DP), lambda i: (i, 0)),
    out_shape=jax.ShapeDtypeStruct((V, DP), jnp.float32),
)


@functools.partial(
    pl.kernel,
    mesh=_mesh,
    compiler_params=_params,
    out_type=jax.ShapeDtypeStruct((BATCH, HIST, DP), jnp.float32),
    scratch_types=[
        pltpu.VMEM((B_PER_W,), jnp.int32),
        pltpu.VMEM((NB, CHUNK, DP), jnp.float32),
        pltpu.SemaphoreType.DMA,
        pltpu.SemaphoreType.DMA,
        pltpu.SemaphoreType.DMA,
        pltpu.SemaphoreType.DMA,
        pltpu.SemaphoreType.DMA,
        pltpu.SemaphoreType.DMA,
        pltpu.SemaphoreType.DMA,
        pltpu.SemaphoreType.DMA,
    ],
)
def _gather(table_hbm, idx_hbm, out_hbm, idx_v, rows_v,
            sg0, sg1, sg2, sg3, so0, so1, so2, so3):
    sem_g = [sg0, sg1, sg2, sg3]
    sem_o = [so0, so1, so2, so3]
    wid = lax.axis_index("s") * NC + lax.axis_index("c")
    base = wid * B_PER_W
    ebase = wid * E_PER_W

    pltpu.sync_copy(idx_hbm.at[pl.ds(base, B_PER_W)], idx_v)

    def g_start(i, b):
        idx_slice = idx_v.at[pl.ds(i * CHUNK, CHUNK)]
        pltpu.async_copy(table_hbm.at[idx_slice], rows_v.at[b], sem_g[b])

    def g_wait(i, b):
        idx_slice = idx_v.at[pl.ds(i * CHUNK, CHUNK)]
        pltpu.make_async_copy(table_hbm.at[idx_slice], rows_v.at[b], sem_g[b]).wait()

    def o_start(i, b):
        e0 = ebase + i * E_PER_CHUNK
        for e in range(E_PER_CHUNK):
            pltpu.async_copy(rows_v.at[b].at[pl.ds(e * HIST, HIST)],
                             out_hbm.at[e0 + e], sem_o[b])

    def o_wait(i, b):
        e0 = ebase + i * E_PER_CHUNK
        for e in range(E_PER_CHUNK):
            pltpu.make_async_copy(rows_v.at[b].at[pl.ds(e * HIST, HIST)],
                                  out_hbm.at[e0 + e], sem_o[b]).wait()

    for b in range(FL):
        g_start(b, b)
    for b in range(NB):
        g_wait(b, b)
        o_start(b, b)
        j = b + FL
        bj = j % NB
        if j >= NB:
            o_wait(j - NB, bj)
        g_start(j, bj)

    def body(g, carry):
        i0 = g * NB
        for b in range(NB):
            i = i0 + b
            g_wait(i, b)
            o_start(i, b)
            j = i + FL
            bj = (b + FL) % NB
            o_wait(j - NB, bj)
            g_start(j, bj)
        return carry

    lax.fori_loop(1, G_OUTER - 1, body, 0)

    i0 = (G_OUTER - 1) * NB
    for b in range(NB):
        i = i0 + b
        g_wait(i, b)
        o_start(i, b)
        j = i + FL
        if j < N_CHUNKS:
            bj = (b + FL) % NB
            o_wait(j - NB, bj)
            g_start(j, bj)
    for b in range(NB):
        o_wait(N_CHUNKS - NB + b, b)


def kernel(idx, table):
    tab_padded = _pad_table_t(table.T)
    out = _gather(tab_padded, idx.reshape(-1))
    return out[:, :, :D]


# speedup vs baseline: 1.3197x; 1.0283x over previous
import functools

import jax
import jax.numpy as jnp
from jax import lax
from jax.experimental import pallas as pl
from jax.experimental.pallas import tpu as pltpu
from jax.experimental.pallas import tpu_sc as plsc

D = 64
DP = 128
V = 1000000
BATCH = 16384
HIST = 50
B = BATCH * HIST
NC, NS = 2, 16
NW = NC * NS
B_PER_W = B // NW
E_PER_W = B_PER_W // HIST

E_PER_CHUNK = 4
CHUNK = E_PER_CHUNK * HIST
N_CHUNKS = B_PER_W // CHUNK
NB = 4
FL = NB // 2
G_OUTER = N_CHUNKS // NB

PAD_BLK = 16384

_mesh = plsc.VectorSubcoreMesh(core_axis_name="c", subcore_axis_name="s")
_params = pltpu.CompilerParams(use_tc_tiling_on_sc=True)


def _pad_body(tab_t_ref, out_ref):
    out_ref[:, :D] = tab_t_ref[...].T


_pad_table_t = pl.pallas_call(
    _pad_body,
    grid=(pl.cdiv(V, PAD_BLK),),
    in_specs=[pl.BlockSpec((D, PAD_BLK), lambda i: (0, i))],
    out_specs=pl.BlockSpec((PAD_BLK, DP), lambda i: (i, 0)),
    out_shape=jax.ShapeDtypeStruct((V, DP), jnp.float32),
)


@functools.partial(
    pl.kernel,
    mesh=_mesh,
    compiler_params=_params,
    out_type=jax.ShapeDtypeStruct((BATCH, HIST, DP), jnp.float32),
    scratch_types=[
        pltpu.VMEM((B_PER_W,), jnp.int32),
        pltpu.VMEM((NB, CHUNK, DP), jnp.float32),
        pltpu.SemaphoreType.DMA,
        pltpu.SemaphoreType.DMA,
        pltpu.SemaphoreType.DMA,
        pltpu.SemaphoreType.DMA,
        pltpu.SemaphoreType.DMA,
        pltpu.SemaphoreType.DMA,
        pltpu.SemaphoreType.DMA,
        pltpu.SemaphoreType.DMA,
    ],
)
def _gather(table_hbm, idx_hbm, out_hbm, idx_v, rows_v,
            sg0, sg1, sg2, sg3, so0, so1, so2, so3):
    sem_g = [sg0, sg1, sg2, sg3]
    sem_o = [so0, so1, so2, so3]
    wid = lax.axis_index("s") * NC + lax.axis_index("c")
    base = wid * B_PER_W
    ebase = wid * E_PER_W

    pltpu.sync_copy(idx_hbm.at[pl.ds(base, B_PER_W)], idx_v)

    def g_start(i, b):
        idx_slice = idx_v.at[pl.ds(i * CHUNK, CHUNK)]
        pltpu.async_copy(table_hbm.at[idx_slice], rows_v.at[b], sem_g[b])

    def g_wait(i, b):
        idx_slice = idx_v.at[pl.ds(i * CHUNK, CHUNK)]
        pltpu.make_async_copy(table_hbm.at[idx_slice], rows_v.at[b], sem_g[b]).wait()

    def o_start(i, b):
        e0 = ebase + i * E_PER_CHUNK
        for e in range(E_PER_CHUNK):
            pltpu.async_copy(rows_v.at[b].at[pl.ds(e * HIST, HIST)],
                             out_hbm.at[e0 + e], sem_o[b])

    def o_wait(i, b):
        e0 = ebase + i * E_PER_CHUNK
        for e in range(E_PER_CHUNK):
            pltpu.make_async_copy(rows_v.at[b].at[pl.ds(e * HIST, HIST)],
                                  out_hbm.at[e0 + e], sem_o[b]).wait()

    for b in range(FL):
        g_start(b, b)
    for b in range(NB):
        g_wait(b, b)
        o_start(b, b)
        j = b + FL
        bj = j % NB
        if j >= NB:
            o_wait(j - NB, bj)
        g_start(j, bj)

    def body(g, carry):
        i0 = g * NB
        for b in range(NB):
            i = i0 + b
            g_wait(i, b)
            o_start(i, b)
            j = i + FL
            bj = (b + FL) % NB
            o_wait(j - NB, bj)
            g_start(j, bj)
        return carry

    lax.fori_loop(1, G_OUTER - 1, body, 0)

    i0 = (G_OUTER - 1) * NB
    for b in range(NB):
        i = i0 + b
        g_wait(i, b)
        o_start(i, b)
        j = i + FL
        if j < N_CHUNKS:
            bj = (b + FL) % NB
            o_wait(j - NB, bj)
            g_start(j, bj)
    for b in range(NB):
        o_wait(N_CHUNKS - NB + b, b)


def kernel(idx, table):
    tab_padded = _pad_table_t(table.T)
    out = _gather(tab_padded, idx.reshape(-1))
    return out[:, :, :D]
